# baseline (speedup 1.0000x reference)
import functools

import jax
import jax.numpy as jnp
from jax import lax
from jax.experimental import pallas as pl
from jax.experimental.pallas import tpu as pltpu
from jax.experimental.pallas import tpu_sc as plsc

DIM = 64
BATCH = 16384
HIST = 50
EVOCAB = 1000000
RVOCAB = 1000

NC = 2
NS = 16
NW = NC * NS
BPW = BATCH // NW
NBLK = BPW // 128

EPAIRS = EVOCAB // 2
RPAIRS = RVOCAB // 2
E_FULL_BLKS = EVOCAB // 128
E_TAIL = EVOCAB - E_FULL_BLKS * 128

_F32 = jnp.float32
_I32 = jnp.int32


def _iota16(off):
    return lax.iota(_I32, 16) + off


def _transpose_block(src_v, dst_v):
    dblk_c = [(_iota16(16 * g) % 64) // 8 for g in range(8)]
    d8_c = [(_iota16(16 * g) % 64) % 8 for g in range(8)]

    def body(pp, carry):
        r0 = 2 * pp
        for g in range(8):
            rloc = r0 + (1 if g >= 4 else 0)
            vals = plsc.load_gather(
                src_v, [dblk_c[g], d8_c[g], jnp.broadcast_to(rloc, (16,))])
            dst_v[pp, pl.ds(16 * g, 16)] = vals
        return carry

    lax.fori_loop(0, 64, body, 0)


def _fmt_kernel(tbl_t, tail_pairs, trow, src_v, dst_v):
    wid = lax.axis_index("s") * NC + lax.axis_index("c")
    start = wid * 244 + jnp.minimum(wid, 4)
    cnt = jnp.where(wid < 4, 245, 244)

    def body(i, carry):
        rblk = start + i
        for dblk in range(8):
            pltpu.sync_copy(
                tbl_t.at[pl.ds(dblk * 8, 8), pl.ds(rblk * 128, 128)],
                src_v.at[dblk])
        _transpose_block(src_v, dst_v)
        pltpu.sync_copy(dst_v, trow.at[pl.ds(rblk * 64, 64)])
        return carry

    lax.fori_loop(0, cnt, body, 0)

    @pl.when(wid == 31)
    def _():
        pltpu.sync_copy(tail_pairs,
                        trow.at[pl.ds(E_FULL_BLKS * 64, E_TAIL // 2)])


def _calc_pidx(idx_v, pidx_v, par_v):
    for i in range(BPW // 16):
        v = idx_v[pl.ds(i * 16, 16)]
        pidx_v[i // 8, pl.ds((i % 8) * 16, 16)] = v >> 1
        par_v[i // 8, pl.ds((i % 8) * 16, 16)] = (v & 1) << 6


def _out_block(src_ref, rows_g, par_g, tb):
    def body(d, carry):
        for g in range(8):
            vals = plsc.load_gather(src_ref, [rows_g[g], par_g[g] + d])
            tb[d, pl.ds(16 * g, 16)] = vals
        return carry

    lax.fori_loop(0, DIM, body, 0)


def _gather_kernel(trow, rrow, eidx_t, ridx_t, oute, outr,
                   rrow_v, idx_v, pidx_v, par_v, buf, tbuf, gsem, ssem):
    wid = lax.axis_index("s") * NC + lax.axis_index("c")
    b0 = wid * BPW
    ivecs = [_iota16(16 * g) for g in range(8)]

    pltpu.sync_copy(rrow, rrow_v)

    def wait_gather(b):
        pltpu.make_async_copy(trow.at[pidx_v.at[0]], buf.at[b], gsem).wait()

    def wait_store(out_hbm, b):
        pltpu.make_async_copy(
            tbuf.at[b], out_hbm.at[0, :, pl.ds(0, 128)], ssem).wait()

    def ent_h(h, carry):
        pltpu.sync_copy(eidx_t.at[h, pl.ds(b0, BPW)], idx_v)
        _calc_pidx(idx_v, pidx_v, par_v)
        pltpu.async_copy(trow.at[pidx_v.at[0]], buf.at[0], gsem)
        for jb in range(NBLK):
            if jb + 1 < NBLK:
                pltpu.async_copy(trow.at[pidx_v.at[jb + 1]],
                                 buf.at[(jb + 1) % 2], gsem)
            wait_gather(jb % 2)
            if jb >= 2:
                wait_store(oute, jb % 2)
            par_g = [par_v[jb, pl.ds(16 * g, 16)] for g in range(8)]
            tb = tbuf.at[jb % 2]
            _out_block(buf.at[jb % 2], ivecs, par_g, tb)
            pltpu.async_copy(
                tb, oute.at[h, :, pl.ds(b0 + jb * 128, 128)], ssem)
        wait_store(oute, 0)
        wait_store(oute, 1)
        return carry

    lax.fori_loop(0, HIST, ent_h, 0)

    def rel_h(h, carry):
        pltpu.sync_copy(ridx_t.at[h, pl.ds(b0, BPW)], idx_v)
        _calc_pidx(idx_v, pidx_v, par_v)
        for jb in range(NBLK):
            if jb >= 2:
                wait_store(outr, jb % 2)
            pidx_g = [pidx_v[jb, pl.ds(16 * g, 16)] for g in range(8)]
            par_g = [par_v[jb, pl.ds(16 * g, 16)] for g in range(8)]
            tb = tbuf.at[jb % 2]
            _out_block(rrow_v, pidx_g, par_g, tb)
            pltpu.async_copy(
                tb, outr.at[h, :, pl.ds(b0 + jb * 128, 128)], ssem)
        wait_store(outr, 0)
        wait_store(outr, 1)
        return carry

    lax.fori_loop(0, HIST, rel_h, 0)


def kernel(entity_indices, relation_indices, entity_table, relation_table):
    mesh = plsc.VectorSubcoreMesh(core_axis_name="c", subcore_axis_name="s")

    tbl_t = jnp.transpose(entity_table)
    eidx_t = jnp.transpose(entity_indices)
    ridx_t = jnp.transpose(relation_indices)

    rel_pairs = relation_table.reshape(RPAIRS, 128)
    tail_pairs = lax.slice(
        entity_table, (E_FULL_BLKS * 128, 0), (EVOCAB, DIM)
    ).reshape(E_TAIL // 2, 128)

    fmt = functools.partial(
        pl.kernel,
        out_type=jax.ShapeDtypeStruct((EPAIRS, 128), _F32),
        mesh=mesh,
        compiler_params=pltpu.CompilerParams(needs_layout_passes=False),
        scratch_types=[
            pltpu.VMEM((8, 8, 128), _F32),
            pltpu.VMEM((64, 128), _F32),
        ],
    )(_fmt_kernel)
    trow = fmt(tbl_t, tail_pairs)

    gat = functools.partial(
        pl.kernel,
        out_type=[jax.ShapeDtypeStruct((HIST, DIM, BATCH), _F32),
                  jax.ShapeDtypeStruct((HIST, DIM, BATCH), _F32)],
        mesh=mesh,
        compiler_params=pltpu.CompilerParams(needs_layout_passes=False),
        scratch_types=[
            pltpu.VMEM((RPAIRS, 128), _F32),
            pltpu.VMEM((BPW,), _I32),
            pltpu.VMEM((NBLK, 128), _I32),
            pltpu.VMEM((NBLK, 128), _I32),
            pltpu.VMEM((2, 128, 128), _F32),
            pltpu.VMEM((2, DIM, 128), _F32),
            pltpu.SemaphoreType.DMA,
            pltpu.SemaphoreType.DMA,
        ],
    )(_gather_kernel)
    oute, outr = gat(trow, rel_pairs, eidx_t, ridx_t)

    return (jnp.transpose(oute, (2, 0, 1)), jnp.transpose(outr, (2, 0, 1)))
